# baseline (speedup 1.0000x reference)
import functools

import jax
import jax.numpy as jnp
from jax import lax
from jax.experimental import pallas as pl
from jax.experimental.pallas import tpu as pltpu
from jax.experimental.pallas import tpu_sc as plsc

_K = 5
_LANES = 16
_GATHER_WORKERS = 8


def _sc_gather_body(pred_flat_hbm, gt_hbm, v_hbm, gt_v, idx_v, val_v, sem):
    n_cores = 2
    wid = lax.axis_index("s") * n_cores + lax.axis_index("c")

    @pl.when(wid < _GATHER_WORKERS)
    def _():
        base = wid * _LANES
        pltpu.async_copy(gt_hbm.at[pl.ds(base, _LANES)], gt_v, sem).wait()
        rows = lax.iota(jnp.int32, _LANES) + base
        idx_v[...] = gt_v[...] + rows * 32768
        pltpu.async_copy(pred_flat_hbm.at[idx_v], val_v, sem).wait()
        pltpu.async_copy(val_v, v_hbm.at[pl.ds(base, _LANES)], sem).wait()


def _make_sc_gather(b, n):
    mesh = plsc.VectorSubcoreMesh(core_axis_name="c", subcore_axis_name="s")
    return pl.kernel(
        _sc_gather_body,
        out_type=jax.ShapeDtypeStruct((b,), jnp.float32),
        mesh=mesh,
        scratch_types=[
            pltpu.VMEM((_LANES,), jnp.int32),
            pltpu.VMEM((_LANES,), jnp.int32),
            pltpu.VMEM((_LANES,), jnp.float32),
            pltpu.SemaphoreType.DMA,
        ],
    )


def _acc_body(gt_ref, v_ref, pred_ref, out_ref):
    i = pl.program_id(0)
    pred = pred_ref[...]
    g = gt_ref[...]
    v = v_ref[...]
    rb, n = pred.shape
    col = jax.lax.broadcasted_iota(jnp.int32, (rb, n), 1)
    cnt_gt = jnp.sum((pred > v).astype(jnp.int32), axis=1)
    cnt_eq = jnp.sum(((pred == v) & (col < g)).astype(jnp.int32), axis=1)
    part = jnp.sum(((cnt_gt + cnt_eq) < _K).astype(jnp.float32)).reshape(1, 1)

    @pl.when(i == 0)
    def _():
        out_ref[...] = jnp.zeros((1, 1), jnp.float32)

    out_ref[...] += part


def kernel(pred, gt):
    b, n = pred.shape
    v = _make_sc_gather(b, n)(pred.reshape(-1), gt)
    rb = 8
    grid = (b // rb,)
    out = pl.pallas_call(
        _acc_body,
        grid=grid,
        in_specs=[
            pl.BlockSpec((rb, 1), lambda i: (i, 0)),
            pl.BlockSpec((rb, 1), lambda i: (i, 0)),
            pl.BlockSpec((rb, n), lambda i: (i, 0)),
        ],
        out_specs=pl.BlockSpec((1, 1), lambda i: (0, 0)),
        out_shape=jax.ShapeDtypeStruct((1, 1), jnp.float32),
    )(gt.reshape(b, 1), v.reshape(b, 1), pred)
    return out[0, 0] / b


# speedup vs baseline: 3.5138x; 3.5138x over previous
import jax
import jax.numpy as jnp
from jax.experimental import pallas as pl

_K = 5


def _acc_body(gt_ref, pred_ref, out_ref):
    i = pl.program_id(0)
    pred = pred_ref[...]
    g = gt_ref[...]
    rb, n = pred.shape
    col = jax.lax.broadcasted_iota(jnp.int32, (rb, n), 1)
    v = jnp.max(jnp.where(col == g, pred, -jnp.inf), axis=1, keepdims=True)
    cnt_gt = jnp.sum((pred > v).astype(jnp.int32), axis=1)
    cnt_eq = jnp.sum((pred == v).astype(jnp.int32), axis=1)

    @pl.when(i == 0)
    def _():
        out_ref[...] = jnp.zeros((1, 1), jnp.float32)

    ambiguous = jnp.any((cnt_gt < _K) & (cnt_gt + cnt_eq - 1 >= _K))

    @pl.when(jnp.logical_not(ambiguous))
    def _():
        part = jnp.sum((cnt_gt < _K).astype(jnp.float32)).reshape(1, 1)
        out_ref[...] += part

    @pl.when(ambiguous)
    def _():
        cnt_eq_low = jnp.sum(((pred == v) & (col < g)).astype(jnp.int32),
                             axis=1)
        part = jnp.sum(((cnt_gt + cnt_eq_low) < _K)
                       .astype(jnp.float32)).reshape(1, 1)
        out_ref[...] += part


def kernel(pred, gt):
    b, n = pred.shape
    rb = 16
    grid = (b // rb,)
    out = pl.pallas_call(
        _acc_body,
        grid=grid,
        in_specs=[
            pl.BlockSpec((rb, 1), lambda i: (i, 0)),
            pl.BlockSpec((rb, n), lambda i: (i, 0)),
        ],
        out_specs=pl.BlockSpec((1, 1), lambda i: (0, 0)),
        out_shape=jax.ShapeDtypeStruct((1, 1), jnp.float32),
    )(gt.reshape(b, 1), pred)
    return out[0, 0] / b
